# baseline (speedup 1.0000x reference)
import math

import jax
import jax.numpy as jnp
from jax import lax
from jax.experimental import pallas as pl
from jax.experimental.pallas import tpu as pltpu
from jax.experimental.pallas import tpu_sc as plsc

VOCAB_SIZE = 1000000
EMB_D = 240
N_TOKENS = 4096 * 200

NUM_CORES = 2
NUM_SUBCORES = 16
NUM_WORKERS = NUM_CORES * NUM_SUBCORES
TOK_PER_WORKER = N_TOKENS // NUM_WORKERS

CHUNK = 128
CHUNKS_PER_WORKER = TOK_PER_WORKER // CHUNK
VECS_PER_ROW = EMB_D // 16

_SCALE = jnp.float32(math.sqrt(EMB_D))


def _emb_body(tok_hbm, table_hbm, out_hbm, idx_v, rows_v, sem):
    wid = lax.axis_index("s") * NUM_CORES + lax.axis_index("c")
    base = wid * TOK_PER_WORKER

    pltpu.sync_copy(tok_hbm.at[pl.ds(wid * CHUNKS_PER_WORKER, CHUNKS_PER_WORKER)],
                    idx_v)

    @pl.loop(0, CHUNKS_PER_WORKER)
    def _chunk(c):
        pltpu.async_copy(table_hbm.at[idx_v.at[c]], rows_v, sem).wait()

        @pl.loop(0, CHUNK)
        def _row(r):
            for j in range(VECS_PER_ROW):
                sl = pl.ds(j * 16, 16)
                rows_v[r, sl] = rows_v[r, sl] * _SCALE

        pltpu.sync_copy(rows_v, out_hbm.at[pl.ds(base + c * CHUNK, CHUNK)])


_emb_call = pl.kernel(
    _emb_body,
    out_type=jax.ShapeDtypeStruct((N_TOKENS, EMB_D), jnp.float32),
    mesh=plsc.VectorSubcoreMesh(core_axis_name="c", subcore_axis_name="s"),
    scratch_types=[
        pltpu.VMEM((CHUNKS_PER_WORKER, CHUNK), jnp.int32),
        pltpu.VMEM((CHUNK, EMB_D), jnp.float32),
        pltpu.SemaphoreType.DMA,
    ],
    compiler_params=pltpu.CompilerParams(use_tc_tiling_on_sc=False),
)


def kernel(tokens, embedding_weight):
    b, s = tokens.shape
    tok = tokens.astype(jnp.int32).reshape(N_TOKENS // CHUNK, CHUNK)
    out = _emb_call(tok, embedding_weight)
    return out.reshape(b, s, EMB_D)


# speedup vs baseline: 3.3083x; 3.3083x over previous
import math

import jax
import jax.numpy as jnp
from jax import lax
from jax.experimental import pallas as pl
from jax.experimental.pallas import tpu as pltpu
from jax.experimental.pallas import tpu_sc as plsc

VOCAB_SIZE = 1000000
EMB_D = 240
ROW_PHYS = 256
N_TOKENS = 4096 * 200

NUM_CORES = 2
NUM_SUBCORES = 16
NUM_WORKERS = NUM_CORES * NUM_SUBCORES
TOK_PER_WORKER = N_TOKENS // NUM_WORKERS
CHUNK = 128
CHUNKS_PER_WORKER = TOK_PER_WORKER // CHUNK
VECS_PER_ROW = EMB_D // 16

_SCALE = math.sqrt(EMB_D)


def _emb_body(tok_hbm, table_hbm, out_hbm, idx_v, buf_g, buf_o, sem_g):
    wid = lax.axis_index("s") * NUM_CORES + lax.axis_index("c")
    base = pl.multiple_of(wid * TOK_PER_WORKER, 8)

    pltpu.sync_copy(tok_hbm.at[pl.ds(base, TOK_PER_WORKER)], idx_v)

    @pl.loop(0, CHUNKS_PER_WORKER)
    def _chunk(c):
        idx = idx_v.at[pl.ds(c * CHUNK, CHUNK)]
        pltpu.async_copy(
            table_hbm.at[idx, pl.ds(0, ROW_PHYS)], buf_g, sem_g).wait()

        @pl.loop(0, CHUNK)
        def _row(r):
            for j in range(VECS_PER_ROW):
                sl = pl.ds(j * 16, 16)
                buf_o[r, sl] = buf_g[r, sl] * _SCALE

        g0 = pl.multiple_of(base + c * CHUNK, 8)
        pltpu.sync_copy(buf_o, out_hbm.at[pl.ds(g0, CHUNK)])


_emb_call = pl.kernel(
    _emb_body,
    out_type=jax.ShapeDtypeStruct((N_TOKENS, EMB_D), jnp.float32),
    mesh=plsc.VectorSubcoreMesh(core_axis_name="c", subcore_axis_name="s"),
    scratch_types=[
        pltpu.VMEM((TOK_PER_WORKER,), jnp.int32),
        pltpu.VMEM((CHUNK, ROW_PHYS), jnp.float32),
        pltpu.VMEM((CHUNK, EMB_D), jnp.float32),
        pltpu.SemaphoreType.DMA,
    ],
)


def kernel(tokens, embedding_weight):
    b, s = tokens.shape
    out = _emb_call(tokens.astype(jnp.int32).reshape(-1), embedding_weight)
    return out.reshape(b, s, EMB_D)


# speedup vs baseline: 3.8612x; 1.1671x over previous
import math

import jax
import jax.numpy as jnp
from jax import lax
from jax.experimental import pallas as pl
from jax.experimental.pallas import tpu as pltpu
from jax.experimental.pallas import tpu_sc as plsc

VOCAB_SIZE = 1000000
EMB_D = 240
ROW_PHYS = 256
N_TOKENS = 4096 * 200

NUM_CORES = 2
NUM_SUBCORES = 16
NUM_WORKERS = NUM_CORES * NUM_SUBCORES
TOK_PER_WORKER = N_TOKENS // NUM_WORKERS
CHUNK = 40
N_CHUNKS = TOK_PER_WORKER // CHUNK
NBUF = 4
N_GROUPS = N_CHUNKS // NBUF
VECS_PER_ROW = EMB_D // 16

_SCALE = math.sqrt(EMB_D)


def _emb_body(tok_hbm, table_hbm, out_hbm,
              idx_v, bg0, bg1, bg2, bg3, bo0, bo1, bo2, bo3,
              sg0, sg1, sg2, sg3, ss0, ss1, ss2, ss3):
    buf_g = (bg0, bg1, bg2, bg3)
    buf_o = (bo0, bo1, bo2, bo3)
    sem_g = (sg0, sg1, sg2, sg3)
    sem_s = (ss0, ss1, ss2, ss3)

    wid = lax.axis_index("s") * NUM_CORES + lax.axis_index("c")
    base = pl.multiple_of(wid * TOK_PER_WORKER, 8)

    pltpu.sync_copy(tok_hbm.at[pl.ds(base, TOK_PER_WORKER)], idx_v)

    def fire_gather(c, j):
        off = pl.multiple_of(c * CHUNK, 8)
        idx = idx_v.at[pl.ds(off, CHUNK)]
        pltpu.async_copy(table_hbm.at[idx, pl.ds(0, ROW_PHYS)],
                         buf_g[j], sem_g[j])

    for j in range(NBUF):
        fire_gather(j, j)

    @pl.loop(0, N_GROUPS)
    def _group(g):
        for j in range(NBUF):
            c = g * NBUF + j
            pltpu.make_async_copy(
                table_hbm.at[idx_v.at[pl.ds(0, CHUNK)], pl.ds(0, ROW_PHYS)],
                buf_g[j], sem_g[j]).wait()
            @pl.when(g >= 1)
            def _():
                pltpu.make_async_copy(
                    buf_o[j], out_hbm.at[pl.ds(0, CHUNK)], sem_s[j]).wait()

            @pl.loop(0, CHUNK)
            def _row(r):
                for v in range(VECS_PER_ROW):
                    sl = pl.ds(v * 16, 16)
                    buf_o[j][r, sl] = buf_g[j][r, sl] * _SCALE

            @pl.when(g < N_GROUPS - 1)
            def _():
                fire_gather(c + NBUF, j)

            g0 = pl.multiple_of(base + c * CHUNK, 8)
            pltpu.async_copy(buf_o[j], out_hbm.at[pl.ds(g0, CHUNK)], sem_s[j])

    for j in range(NBUF):
        pltpu.make_async_copy(
            buf_o[j], out_hbm.at[pl.ds(0, CHUNK)], sem_s[j]).wait()


_emb_call = pl.kernel(
    _emb_body,
    out_type=jax.ShapeDtypeStruct((N_TOKENS, EMB_D), jnp.float32),
    mesh=plsc.VectorSubcoreMesh(core_axis_name="c", subcore_axis_name="s"),
    scratch_types=(
        [pltpu.VMEM((TOK_PER_WORKER,), jnp.int32)]
        + [pltpu.VMEM((CHUNK, ROW_PHYS), jnp.float32) for _ in range(NBUF)]
        + [pltpu.VMEM((CHUNK, EMB_D), jnp.float32) for _ in range(NBUF)]
        + [pltpu.SemaphoreType.DMA for _ in range(2 * NBUF)]
    ),
)


def kernel(tokens, embedding_weight):
    b, s = tokens.shape
    out = _emb_call(tokens.astype(jnp.int32).reshape(-1), embedding_weight)
    return out.reshape(b, s, EMB_D)
